# baseline (speedup 1.0000x reference)
import functools

import jax
import jax.numpy as jnp
import numpy as np
from jax.experimental import pallas as pl
from jax.experimental.pallas import tpu as pltpu
from jax.experimental.pallas import tpu_sc as plsc

CLUSTERS = 128
ITERATIONS = 10
NBITS = 32
TOPK = 32

NEG_INF = float("-inf")

HI = jax.lax.Precision.HIGHEST


def _dot(a, b, precision=None):
    return jax.lax.dot_general(a, b, (((1,), (0,)), ((), ())),
                               precision=precision,
                               preferred_element_type=jnp.float32)


def _dot_t(a, b, precision=None):
    return jax.lax.dot_general(a, b, (((1,), (1,)), ((), ())),
                               precision=precision,
                               preferred_element_type=jnp.float32)


def _split_bf16(x):
    hi = x.astype(jnp.bfloat16)
    lo = (x - hi.astype(jnp.float32)).astype(jnp.bfloat16)
    return hi, lo


def _dot_t_3x(a, b):
    ah, al = _split_bf16(a)
    bh, bl = _split_bf16(b)
    return (_dot_t(ah, bh) + _dot_t(ah, bl)) + _dot_t(al, bh)


def _stage_a_body(q_ref, k_ref, v_ref, key_ref, pwt_ref, pb_ref,
                  ac_ref, tkg_ref, tkv_ref, vbp_ref):
    L, E = q_ref.shape[1], q_ref.shape[2]
    S = k_ref.shape[1]
    C = CLUSTERS
    Q = q_ref[0]
    K = k_ref[0]
    V = v_ref[0]
    key_row = key_ref[0]
    temp = jnp.float32(1.0 / np.sqrt(E))

    proj = _dot(Q, pwt_ref[...]) + pb_ref[0:1, :]
    bits = (proj > 0).astype(jnp.float32)
    e_bf = (1.0 - 2.0 * bits).astype(jnp.bfloat16)

    c_sub = jax.lax.broadcasted_iota(jnp.int32, (C, L), 0)
    l_lane = jax.lax.broadcasted_iota(jnp.int32, (C, L), 1)
    init_col = (c_sub * (L - 1)) // (C - 1)
    ig = (l_lane == init_col).astype(jnp.float32)
    cent0 = _dot(ig, bits)

    c_sub_f = c_sub.astype(jnp.float32)

    def assign_row_of(cent_bf):
        dT = _dot_t(cent_bf, e_bf)
        key = (dT + (NBITS + 1)) * C + c_sub_f
        kmin = jnp.min(key, axis=0, keepdims=True)
        return kmin.astype(jnp.int32) & (C - 1)

    bits_aug = jnp.concatenate(
        [bits, jnp.ones((L, NBITS), jnp.float32)], axis=1
    ).astype(jnp.bfloat16)

    def lloyd_iter(_, cent_bf):
        amin = assign_row_of(cent_bf)
        ohT = (c_sub == amin).astype(jnp.bfloat16)
        sums_aug = _dot(ohT, bits_aug)
        sums = sums_aug[:, :NBITS]
        counts = sums_aug[:, NBITS:NBITS + 1]
        new_cent = (sums / jnp.maximum(counts, 1.0) >= 0.5)
        return jnp.where(counts > 0,
                         new_cent.astype(jnp.bfloat16), cent_bf)

    cent = jax.lax.fori_loop(0, ITERATIONS, lloyd_iter,
                             cent0.astype(jnp.bfloat16))
    amin_row = assign_row_of(cent)
    ohT = (c_sub == amin_row).astype(jnp.float32)

    q_aug = jnp.concatenate(
        [Q, jnp.ones((L, E), jnp.float32)], axis=1)
    seg = _dot(ohT, q_aug, HI)
    counts = jnp.maximum(seg[:, E:E + 1], 1.0)
    qg = seg[:, :E] * (1.0 / counts)

    qk = _dot_t(qg, K) + key_row

    NG = 4
    RG = C // NG
    s_iota_g = jax.lax.broadcasted_iota(jnp.int32, (RG, S), 1)
    k_iota_g = jax.lax.broadcasted_iota(jnp.int32, (RG, TOPK), 1)
    w_parts = [qk[RG * g:RG * (g + 1), :] for g in range(NG)]
    tki_parts = [jnp.zeros((RG, TOPK), jnp.int32) for _ in range(NG)]
    tkv_parts = [jnp.zeros((RG, TOPK), jnp.float32) for _ in range(NG)]
    for kk in range(TOPK):
        for g in range(NG):
            w = w_parts[g]
            m = jnp.max(w, axis=1, keepdims=True)
            idx = jnp.min(jnp.where(w == m, s_iota_g, S), axis=1,
                          keepdims=True)
            w_parts[g] = jnp.where(s_iota_g == idx, NEG_INF, w)
            tki_parts[g] = tki_parts[g] + jnp.where(k_iota_g == kk, idx, 0)
            tkv_parts[g] = tkv_parts[g] + jnp.where(k_iota_g == kk, m, 0.0)
    w = jnp.concatenate(w_parts, axis=0)
    tki = jnp.concatenate(tki_parts, axis=0)
    tkv = jnp.concatenate(tkv_parts, axis=0)

    z = temp * qk
    zmax = jnp.max(z, axis=1, keepdims=True)
    p = jnp.exp(z - zmax)
    denom = jnp.sum(p, axis=1, keepdims=True)
    a_masked = jnp.where(w == NEG_INF, 0.0, p) / denom
    a_bottomk = jnp.sum(a_masked, axis=1, keepdims=True)
    vb = _dot(a_masked, V)

    e_lane = jax.lax.broadcasted_iota(jnp.int32, (C, E), 1)
    a_pad = jnp.where(e_lane == 0, a_bottomk, 0.0)
    vbp = jnp.concatenate([vb, a_pad], axis=1)

    amin_col = jnp.transpose(amin_row)
    ac_ref[0] = jnp.broadcast_to(amin_col, (L, 8)).astype(jnp.int32)
    tkg_ref[0] = tki + pl.program_id(0) * S
    tkv_ref[0] = tkv
    vbp_ref[0] = vbp


def _gather_rows(kvflat, idx):
    B = idx.shape[0]
    D = kvflat.shape[1]
    info = plsc.get_sparse_core_info()
    NC, NS = info.num_cores, info.num_subcores
    NW = NC * NS
    b_per_w = B // NW
    chunk = 128
    nchunk = b_per_w // chunk
    mesh = plsc.VectorSubcoreMesh(core_axis_name="c", subcore_axis_name="s")

    @functools.partial(
        pl.kernel, mesh=mesh,
        out_type=jax.ShapeDtypeStruct((B, D), jnp.float32),
        scratch_types=[
            pltpu.VMEM((chunk,), jnp.int32),
            pltpu.VMEM((chunk, D), jnp.float32),
            pltpu.SemaphoreType.DMA,
        ],
    )
    def gath(kv_hbm, idx_hbm, out_hbm, idx_v, rows, sem):
        wid = jax.lax.axis_index("s") * NC + jax.lax.axis_index("c")
        base = wid * b_per_w
        for j in range(nchunk):
            off = base + j * chunk
            pltpu.sync_copy(idx_hbm.at[pl.ds(off, chunk)], idx_v)
            pltpu.async_copy(kv_hbm.at[idx_v], rows, sem).wait()
            pltpu.sync_copy(rows, out_hbm.at[pl.ds(off, chunk)])

    return gath(kvflat, idx)


def _stage_c_body(q_ref, kvg_ref, ac_ref, tkv0_ref, vbp_ref, out_ref):
    L, E = q_ref.shape[1], q_ref.shape[2]
    C = CLUSTERS
    CH = 128
    NCH = C // CH
    W = CH * TOPK
    temp = jnp.float32(1.0 / np.sqrt(E))

    Q = q_ref[0]
    qh, ql = _split_bf16(Q)
    assign = ac_ref[0][:, 0:1]
    vbp = vbp_ref[0]

    c_iota = jax.lax.broadcasted_iota(jnp.int32, (L, C), 1)
    o128 = (c_iota == assign).astype(jnp.bfloat16)
    vbh, vbl = _split_bf16(vbp)
    r = _dot(jnp.concatenate([o128, o128], axis=1),
             jnp.concatenate([vbh, vbl], axis=0))
    v_bottom = r[:, :E]
    a_bk = r[:, E:E + 1]

    inf_row = jnp.isinf(tkv0_ref[0][0:1, :])

    j_iota = jax.lax.broadcasted_iota(jnp.int32, (L, W), 1)
    jdiv = j_iota // TOPK

    qcat = jnp.concatenate([qh, qh, ql], axis=1)

    def pass1(ch, qk_t):
        kg = kvg_ref[0, pl.ds(ch * W, W), :E]
        kh, kl = _split_bf16(kg)
        kcat = jnp.concatenate([kh, kl, kh], axis=1)
        t1 = _dot_t(qcat, kcat)
        obig = (jdiv + (CH * ch) == assign).astype(jnp.float32)
        s = t1 * obig
        while s.shape[1] > TOPK:
            half = s.shape[1] // 2
            s = s[:, :half] + s[:, half:]
        return qk_t + s

    qk_t = jax.lax.fori_loop(0, NCH, pass1, jnp.zeros((L, TOPK), jnp.float32))

    qk_t = jnp.where(inf_row, NEG_INF, qk_t)
    z = temp * qk_t
    zmax = jnp.max(z, axis=1, keepdims=True)
    p = jnp.exp(z - zmax)
    a_t = p / jnp.sum(p, axis=1, keepdims=True)
    a_t = a_t * (1.0 - a_bk)

    ah, al = _split_bf16(a_t)
    arh = pltpu.repeat(ah, W // TOPK, axis=1)
    arl = pltpu.repeat(al, W // TOPK, axis=1)

    def pass2(ch, out):
        obig = (jdiv + (CH * ch) == assign).astype(jnp.bfloat16)
        bh = arh * obig
        bl = arl * obig
        vg = kvg_ref[0, pl.ds(ch * W, W), E:]
        vh, vl = _split_bf16(vg)
        return out + ((_dot(bh, vh) + _dot(bh, vl)) + _dot(bl, vh))

    out_ref[0] = jax.lax.fori_loop(0, NCH, pass2, v_bottom)


def _run(Q, K, V, key_row, pwt, pb, heads_per_batch):
    NH, L, E = Q.shape
    S = K.shape[1]
    C = CLUSTERS
    G = C * TOPK
    hmap = lambda h: (h, 0, 0)
    zmap = lambda h: (0, 0)
    stage_a = pl.pallas_call(
        _stage_a_body,
        grid=(NH,),
        in_specs=[
            pl.BlockSpec((1, L, E), hmap),
            pl.BlockSpec((1, S, E), hmap),
            pl.BlockSpec((1, S, E), hmap),
            pl.BlockSpec((1, 1, S), hmap),
            pl.BlockSpec((E, NBITS), zmap),
            pl.BlockSpec((8, NBITS), zmap),
        ],
        out_specs=[
            pl.BlockSpec((1, L, 8), hmap),
            pl.BlockSpec((1, C, TOPK), hmap),
            pl.BlockSpec((1, C, TOPK), hmap),
            pl.BlockSpec((1, C, 2 * E), hmap),
        ],
        out_shape=[
            jax.ShapeDtypeStruct((NH, L, 8), jnp.int32),
            jax.ShapeDtypeStruct((NH, C, TOPK), jnp.int32),
            jax.ShapeDtypeStruct((NH, C, TOPK), jnp.float32),
            jax.ShapeDtypeStruct((NH, C, 2 * E), jnp.float32),
        ],
    )
    ac, tkg, tkv, vbp = stage_a(Q, K, V, key_row, pwt, pb)

    kv = jnp.concatenate([K, V], axis=2).reshape(NH * S, 2 * E)
    kvg = _gather_rows(kv, tkg.reshape(NH * G)).reshape(NH, G, 2 * E)

    head0map = lambda h: ((h // heads_per_batch) * heads_per_batch, 0, 0)
    stage_c = pl.pallas_call(
        _stage_c_body,
        grid=(NH,),
        in_specs=[
            pl.BlockSpec((1, L, E), hmap),
            pl.BlockSpec((1, G, 2 * E), hmap),
            pl.BlockSpec((1, L, 8), hmap),
            pl.BlockSpec((1, C, TOPK), head0map),
            pl.BlockSpec((1, C, 2 * E), hmap),
        ],
        out_specs=pl.BlockSpec((1, L, E), hmap),
        out_shape=jax.ShapeDtypeStruct((NH, L, E), jnp.float32),
        compiler_params=pltpu.CompilerParams(
            vmem_limit_bytes=100 * 1024 * 1024),
    )
    return stage_c(Q, kvg, ac, tkv, vbp)


def kernel(queries, keys, values, key_lengths_additive, planes):
    N, L, H, E = queries.shape
    S = keys.shape[1]
    Q = jnp.transpose(queries, (0, 2, 1, 3)).reshape(N * H, L, E)
    K = jnp.transpose(keys, (0, 2, 1, 3)).reshape(N * H, S, E)
    V = jnp.transpose(values, (0, 2, 1, 3)).reshape(N * H, S, E)
    key_row = jnp.broadcast_to(
        key_lengths_additive[:, None, None, :], (N, H, 1, S)
    ).reshape(N * H, 1, S)
    pwt = planes[:, :E].T
    pb = jnp.zeros((8, NBITS), jnp.float32).at[0].set(planes[:, E])
    out = _run(Q, K, V, key_row, pwt, pb, H)
    return jnp.transpose(out.reshape(N, H, L, E), (0, 2, 1, 3))


# speedup vs baseline: 1.1064x; 1.1064x over previous
import functools

import jax
import jax.numpy as jnp
import numpy as np
from jax.experimental import pallas as pl
from jax.experimental.pallas import tpu as pltpu
from jax.experimental.pallas import tpu_sc as plsc

CLUSTERS = 128
ITERATIONS = 10
NBITS = 32
TOPK = 32

NEG_INF = float("-inf")

HI = jax.lax.Precision.HIGHEST


def _dot(a, b, precision=None):
    return jax.lax.dot_general(a, b, (((1,), (0,)), ((), ())),
                               precision=precision,
                               preferred_element_type=jnp.float32)


def _dot_t(a, b, precision=None):
    return jax.lax.dot_general(a, b, (((1,), (1,)), ((), ())),
                               precision=precision,
                               preferred_element_type=jnp.float32)


def _split_bf16(x):
    hi = x.astype(jnp.bfloat16)
    lo = (x - hi.astype(jnp.float32)).astype(jnp.bfloat16)
    return hi, lo


def _dot_t_3x(a, b):
    ah, al = _split_bf16(a)
    bh, bl = _split_bf16(b)
    return (_dot_t(ah, bh) + _dot_t(ah, bl)) + _dot_t(al, bh)


def _stage_a_body(q_ref, k_ref, v_ref, key_ref, pwt_ref, pb_ref,
                  ac_ref, tkg_ref, tkv_ref, vbp_ref):
    L, E = q_ref.shape[1], q_ref.shape[2]
    S = k_ref.shape[1]
    C = CLUSTERS
    Q = q_ref[0]
    K = k_ref[0]
    V = v_ref[0]
    key_row = key_ref[0]
    temp = jnp.float32(1.0 / np.sqrt(E))

    proj = _dot(Q, pwt_ref[...]) + pb_ref[0:1, :]
    bits = (proj > 0).astype(jnp.float32)
    e_bf = (1.0 - 2.0 * bits).astype(jnp.bfloat16)

    c_sub = jax.lax.broadcasted_iota(jnp.int32, (C, L), 0)
    l_lane = jax.lax.broadcasted_iota(jnp.int32, (C, L), 1)
    init_col = (c_sub * (L - 1)) // (C - 1)
    ig = (l_lane == init_col).astype(jnp.float32)
    cent0 = _dot(ig, bits)

    c_sub_f = c_sub.astype(jnp.float32)

    def assign_row_of(cent_bf):
        dT = _dot_t(cent_bf, e_bf)
        key = (dT + (NBITS + 1)) * C + c_sub_f
        kmin = jnp.min(key, axis=0, keepdims=True)
        return kmin.astype(jnp.int32) & (C - 1)

    bits_aug = jnp.concatenate(
        [bits, jnp.ones((L, NBITS), jnp.float32)], axis=1
    ).astype(jnp.bfloat16)

    def lloyd_iter(_, cent_bf):
        amin = assign_row_of(cent_bf)
        ohT = (c_sub == amin).astype(jnp.bfloat16)
        sums_aug = _dot(ohT, bits_aug)
        sums = sums_aug[:, :NBITS]
        counts = sums_aug[:, NBITS:NBITS + 1]
        new_cent = (sums / jnp.maximum(counts, 1.0) >= 0.5)
        return jnp.where(counts > 0,
                         new_cent.astype(jnp.bfloat16), cent_bf)

    cent = jax.lax.fori_loop(0, ITERATIONS, lloyd_iter,
                             cent0.astype(jnp.bfloat16))
    amin_row = assign_row_of(cent)
    ohT = (c_sub == amin_row).astype(jnp.float32)

    q_aug = jnp.concatenate(
        [Q, jnp.ones((L, E), jnp.float32)], axis=1)
    seg = _dot(ohT, q_aug, HI)
    counts = jnp.maximum(seg[:, E:E + 1], 1.0)
    qg = seg[:, :E] * (1.0 / counts)

    qk = _dot_t(qg, K) + key_row

    NG = 4
    RG = C // NG
    s_iota_g = jax.lax.broadcasted_iota(jnp.int32, (RG, S), 1)
    k_iota_g = jax.lax.broadcasted_iota(jnp.int32, (RG, TOPK), 1)
    w_parts = [qk[RG * g:RG * (g + 1), :] for g in range(NG)]
    tki_parts = [jnp.zeros((RG, TOPK), jnp.int32) for _ in range(NG)]
    tkv_parts = [jnp.zeros((RG, TOPK), jnp.float32) for _ in range(NG)]
    for kk in range(TOPK):
        for g in range(NG):
            w = w_parts[g]
            m = jnp.max(w, axis=1, keepdims=True)
            idx = jnp.min(jnp.where(w == m, s_iota_g, S), axis=1,
                          keepdims=True)
            w_parts[g] = jnp.where(s_iota_g == idx, NEG_INF, w)
            tki_parts[g] = tki_parts[g] + jnp.where(k_iota_g == kk, idx, 0)
            tkv_parts[g] = tkv_parts[g] + jnp.where(k_iota_g == kk, m, 0.0)
    w = jnp.concatenate(w_parts, axis=0)
    tki = jnp.concatenate(tki_parts, axis=0)
    tkv = jnp.concatenate(tkv_parts, axis=0)

    z = temp * qk
    zmax = jnp.max(z, axis=1, keepdims=True)
    p = jnp.exp(z - zmax)
    denom = jnp.sum(p, axis=1, keepdims=True)
    a_masked = jnp.where(w == NEG_INF, 0.0, p) / denom
    a_bottomk = jnp.sum(a_masked, axis=1, keepdims=True)
    vb = _dot(a_masked, V)

    e_lane = jax.lax.broadcasted_iota(jnp.int32, (C, E), 1)
    a_pad = jnp.where(e_lane == 0, a_bottomk, 0.0)
    vbp = jnp.concatenate([vb, a_pad], axis=1)

    amin_col = jnp.transpose(amin_row)
    ac_ref[0] = jnp.broadcast_to(amin_col, (L, 8)).astype(jnp.int32)
    tkg_ref[0] = tki + pl.program_id(0) * S
    tkv_ref[0] = tkv
    vbp_ref[0] = vbp


def _gather_rows(kvflat, idx):
    B = idx.shape[0]
    D = kvflat.shape[1]
    info = plsc.get_sparse_core_info()
    NC, NS = info.num_cores, info.num_subcores
    NW = NC * NS
    b_per_w = B // NW
    chunk = 128
    nchunk = b_per_w // chunk
    mesh = plsc.VectorSubcoreMesh(core_axis_name="c", subcore_axis_name="s")

    @functools.partial(
        pl.kernel, mesh=mesh,
        out_type=jax.ShapeDtypeStruct((B, D), jnp.float32),
        scratch_types=[
            pltpu.VMEM((chunk,), jnp.int32),
            pltpu.VMEM((chunk, D), jnp.float32),
            pltpu.SemaphoreType.DMA,
        ],
    )
    def gath(kv_hbm, idx_hbm, out_hbm, idx_v, rows, sem):
        wid = jax.lax.axis_index("s") * NC + jax.lax.axis_index("c")
        base = wid * b_per_w
        for j in range(nchunk):
            off = base + j * chunk
            pltpu.sync_copy(idx_hbm.at[pl.ds(off, chunk)], idx_v)
            pltpu.async_copy(kv_hbm.at[idx_v], rows, sem).wait()
            pltpu.sync_copy(rows, out_hbm.at[pl.ds(off, chunk)])

    return gath(kvflat, idx)


def _stage_c_body(q_ref, kvg_ref, ac_ref, tkv0_ref, vbp_ref, out_ref):
    L, E = q_ref.shape[1], q_ref.shape[2]
    C = CLUSTERS
    CH = 64
    NCH = C // CH
    W = CH * TOPK
    temp = jnp.float32(1.0 / np.sqrt(E))

    Q = q_ref[0]
    qh, ql = _split_bf16(Q)
    assign = ac_ref[0][:, 0:1]
    vbp = vbp_ref[0]

    c_iota = jax.lax.broadcasted_iota(jnp.int32, (L, C), 1)
    o128 = (c_iota == assign).astype(jnp.bfloat16)
    vbh, vbl = _split_bf16(vbp)
    r = _dot(jnp.concatenate([o128, o128], axis=1),
             jnp.concatenate([vbh, vbl], axis=0))
    v_bottom = r[:, :E]
    a_bk = r[:, E:E + 1]

    inf_row = jnp.isinf(tkv0_ref[0][0:1, :])

    j_iota = jax.lax.broadcasted_iota(jnp.int32, (L, W), 1)
    jdiv = j_iota // TOPK

    qcat = jnp.concatenate([qh, qh, ql], axis=1)

    def pass1(ch, qk_t):
        kg = kvg_ref[0, pl.ds(ch * W, W), :E]
        kh, kl = _split_bf16(kg)
        kcat = jnp.concatenate([kh, kl, kh], axis=1)
        t1 = _dot_t(qcat, kcat)
        obig = (jdiv + (CH * ch) == assign).astype(jnp.float32)
        s = t1 * obig
        while s.shape[1] > TOPK:
            half = s.shape[1] // 2
            s = s[:, :half] + s[:, half:]
        return qk_t + s

    qk_t = jax.lax.fori_loop(0, NCH, pass1, jnp.zeros((L, TOPK), jnp.float32))

    qk_t = jnp.where(inf_row, NEG_INF, qk_t)
    z = temp * qk_t
    zmax = jnp.max(z, axis=1, keepdims=True)
    p = jnp.exp(z - zmax)
    a_t = p / jnp.sum(p, axis=1, keepdims=True)
    a_t = a_t * (1.0 - a_bk)

    ah, al = _split_bf16(a_t)
    arh = pltpu.repeat(ah, W // TOPK, axis=1)
    arl = pltpu.repeat(al, W // TOPK, axis=1)

    def pass2(ch, out):
        obig = (jdiv + (CH * ch) == assign).astype(jnp.bfloat16)
        bh = arh * obig
        bl = arl * obig
        vg = kvg_ref[0, pl.ds(ch * W, W), E:]
        vh, vl = _split_bf16(vg)
        return out + ((_dot(bh, vh) + _dot(bh, vl)) + _dot(bl, vh))

    out_ref[0] = jax.lax.fori_loop(0, NCH, pass2, v_bottom)


def _run(Q, K, V, key_row, pwt, pb, heads_per_batch):
    NH, L, E = Q.shape
    S = K.shape[1]
    C = CLUSTERS
    G = C * TOPK
    hmap = lambda h: (h, 0, 0)
    zmap = lambda h: (0, 0)
    stage_a = pl.pallas_call(
        _stage_a_body,
        grid=(NH,),
        in_specs=[
            pl.BlockSpec((1, L, E), hmap),
            pl.BlockSpec((1, S, E), hmap),
            pl.BlockSpec((1, S, E), hmap),
            pl.BlockSpec((1, 1, S), hmap),
            pl.BlockSpec((E, NBITS), zmap),
            pl.BlockSpec((8, NBITS), zmap),
        ],
        out_specs=[
            pl.BlockSpec((1, L, 8), hmap),
            pl.BlockSpec((1, C, TOPK), hmap),
            pl.BlockSpec((1, C, TOPK), hmap),
            pl.BlockSpec((1, C, 2 * E), hmap),
        ],
        out_shape=[
            jax.ShapeDtypeStruct((NH, L, 8), jnp.int32),
            jax.ShapeDtypeStruct((NH, C, TOPK), jnp.int32),
            jax.ShapeDtypeStruct((NH, C, TOPK), jnp.float32),
            jax.ShapeDtypeStruct((NH, C, 2 * E), jnp.float32),
        ],
    )
    ac, tkg, tkv, vbp = stage_a(Q, K, V, key_row, pwt, pb)

    kv = jnp.concatenate([K, V], axis=2).reshape(NH * S, 2 * E)
    kvg = _gather_rows(kv, tkg.reshape(NH * G)).reshape(NH, G, 2 * E)

    head0map = lambda h: ((h // heads_per_batch) * heads_per_batch, 0, 0)
    stage_c = pl.pallas_call(
        _stage_c_body,
        grid=(NH,),
        in_specs=[
            pl.BlockSpec((1, L, E), hmap),
            pl.BlockSpec((1, G, 2 * E), hmap),
            pl.BlockSpec((1, L, 8), hmap),
            pl.BlockSpec((1, C, TOPK), head0map),
            pl.BlockSpec((1, C, 2 * E), hmap),
        ],
        out_specs=pl.BlockSpec((1, L, E), hmap),
        out_shape=jax.ShapeDtypeStruct((NH, L, E), jnp.float32),
        compiler_params=pltpu.CompilerParams(
            vmem_limit_bytes=100 * 1024 * 1024),
    )
    return stage_c(Q, kvg, ac, tkv, vbp)


def kernel(queries, keys, values, key_lengths_additive, planes):
    N, L, H, E = queries.shape
    S = keys.shape[1]
    Q = jnp.transpose(queries, (0, 2, 1, 3)).reshape(N * H, L, E)
    K = jnp.transpose(keys, (0, 2, 1, 3)).reshape(N * H, S, E)
    V = jnp.transpose(values, (0, 2, 1, 3)).reshape(N * H, S, E)
    key_row = jnp.broadcast_to(
        key_lengths_additive[:, None, None, :], (N, H, 1, S)
    ).reshape(N * H, 1, S)
    pwt = planes[:, :E].T
    pb = jnp.zeros((8, NBITS), jnp.float32).at[0].set(planes[:, E])
    out = _run(Q, K, V, key_row, pwt, pb, H)
    return jnp.transpose(out.reshape(N, H, L, E), (0, 2, 1, 3))
